# baseline (speedup 1.0000x reference)
import functools

import jax
import jax.numpy as jnp
from jax import lax
from jax.experimental import pallas as pl
from jax.experimental.pallas import tpu as pltpu
from jax.experimental.pallas import tpu_sc as plsc

NUM_CORES = 2
NUM_SUBCORES = 16
NUM_WORKERS = NUM_CORES * NUM_SUBCORES

CHUNK = 32
NBUF = 3


def _make_gather(B: int, V: int, D: int):
    b_per_w = B // NUM_WORKERS
    n_chunks = b_per_w // CHUNK
    mesh = plsc.VectorSubcoreMesh(core_axis_name="c", subcore_axis_name="s")

    @functools.partial(
        pl.kernel,
        mesh=mesh,
        out_type=jax.ShapeDtypeStruct((B, D), jnp.float32),
        scratch_types=(
            [pltpu.VMEM((b_per_w,), jnp.int32)]
            + [pltpu.VMEM((CHUNK, D), jnp.float32)] * NBUF
            + [pltpu.SemaphoreType.DMA] * (2 * NBUF)
        ),
    )
    def gather_kernel(pos_hbm, table_hbm, out_hbm, idx_v, *rest):
        bufs = rest[:NBUF]
        gsems = rest[NBUF:2 * NBUF]
        wsems = rest[2 * NBUF:]
        wid = lax.axis_index("s") * NUM_CORES + lax.axis_index("c")
        base = wid * b_per_w

        pltpu.sync_copy(pos_hbm.at[pl.ds(base, b_per_w)], idx_v)

        def gather_desc(j, b):
            return pltpu.make_async_copy(
                table_hbm.at[idx_v.at[pl.ds(j * CHUNK, CHUNK)]],
                bufs[b], gsems[b])

        def wb_desc(j, b):
            return pltpu.make_async_copy(
                bufs[b], out_hbm.at[pl.ds(base + j * CHUNK, CHUNK)],
                wsems[b])

        def step(i, b, wait_wb):
            if wait_wb:
                wb_desc(i - NBUF, b).wait()
            gather_desc(i, b).start()
            b2 = (b - 1) % NBUF
            gather_desc(i - 1, b2).wait()
            wb_desc(i - 1, b2).start()

        gather_desc(0, 0).start()
        step(1, 1, wait_wb=False)
        step(2, 2, wait_wb=False)

        main = n_chunks - NBUF
        n_k = (main - NBUF) // NBUF + 1

        def body(k, carry):
            i0 = NBUF + k * NBUF
            for db in range(NBUF):
                step(i0 + db, db, wait_wb=True)
            return carry

        lax.fori_loop(0, n_k, body, 0)

        for i in range(NBUF + n_k * NBUF, n_chunks):
            step(i, i % NBUF, wait_wb=True)

        j = n_chunks - 1
        gather_desc(j, j % NBUF).wait()
        wb_desc(j, j % NBUF).start()

        for j in range(n_chunks - NBUF, n_chunks):
            wb_desc(j, j % NBUF).wait()

    return gather_kernel


def kernel(pos, table):
    V, D = table.shape
    flat_pos = pos.reshape(-1).astype(jnp.int32)
    B = flat_pos.shape[0]
    h = B // 2
    g = _make_gather(h, V, D)
    out1 = g(flat_pos[:h], table)
    out2 = g(flat_pos[h:], table)
    out = jnp.concatenate([out1, out2], axis=0)
    return out.reshape(pos.shape + (D,))


# speedup vs baseline: 1.8297x; 1.8297x over previous
import functools

import jax
import jax.numpy as jnp
from jax import lax
from jax.experimental import pallas as pl
from jax.experimental.pallas import tpu as pltpu
from jax.experimental.pallas import tpu_sc as plsc

NUM_CORES = 2
NUM_SUBCORES = 16
NUM_WORKERS = NUM_CORES * NUM_SUBCORES

CHUNK = 16
NBUF = 6


def _make_gather(B: int, V: int, D: int):
    b_per_w = B // NUM_WORKERS
    n_chunks = b_per_w // CHUNK
    mesh = plsc.VectorSubcoreMesh(core_axis_name="c", subcore_axis_name="s")

    @functools.partial(
        pl.kernel,
        mesh=mesh,
        out_type=jax.ShapeDtypeStruct((B, D), jnp.float32),
        scratch_types=(
            [pltpu.VMEM((b_per_w,), jnp.int32)]
            + [pltpu.VMEM((CHUNK, D), jnp.float32)] * NBUF
            + [pltpu.SemaphoreType.DMA] * (2 * NBUF)
        ),
    )
    def gather_kernel(pos_hbm, table_hbm, out_hbm, idx_v, *rest):
        bufs = rest[:NBUF]
        gsems = rest[NBUF:2 * NBUF]
        wsems = rest[2 * NBUF:]
        wid = lax.axis_index("s") * NUM_CORES + lax.axis_index("c")
        base = wid * b_per_w

        pltpu.sync_copy(pos_hbm.at[pl.ds(base, b_per_w)], idx_v)

        def gather_desc(j, b):
            return pltpu.make_async_copy(
                table_hbm.at[idx_v.at[pl.ds(j * CHUNK, CHUNK)]],
                bufs[b], gsems[b])

        def wb_desc(j, b):
            return pltpu.make_async_copy(
                bufs[b], out_hbm.at[pl.ds(base + j * CHUNK, CHUNK)],
                wsems[b])

        for b in range(NBUF):
            gather_desc(b, b).start()

        def step(j, b):
            gather_desc(j, b).wait()
            wb_desc(j, b).start()
            wb_desc(j, b).wait()
            gather_desc(j + NBUF, b).start()

        main = n_chunks - NBUF
        unrolled = (main // NBUF) * NBUF

        def body(k, carry):
            for b in range(NBUF):
                step(k * NBUF + b, b)
            return carry

        lax.fori_loop(0, main // NBUF, body, 0)

        for j in range(unrolled, main):
            step(j, j % NBUF)

        for j in range(main, n_chunks):
            gather_desc(j, j % NBUF).wait()
            wb_desc(j, j % NBUF).start()
        for j in range(main, n_chunks):
            wb_desc(j, j % NBUF).wait()

    return gather_kernel


def kernel(pos, table):
    V, D = table.shape
    flat_pos = pos.reshape(-1).astype(jnp.int32)
    B = flat_pos.shape[0]
    out = _make_gather(B, V, D)(flat_pos, table)
    return out.reshape(pos.shape + (D,))


# speedup vs baseline: 1.8325x; 1.0016x over previous
import functools

import jax
import jax.numpy as jnp
from jax import lax
from jax.experimental import pallas as pl
from jax.experimental.pallas import tpu as pltpu
from jax.experimental.pallas import tpu_sc as plsc

NUM_CORES = 2
NUM_SUBCORES = 16
NUM_WORKERS = NUM_CORES * NUM_SUBCORES

CHUNK = 32
NBUF = 3


def _make_gather(B: int, V: int, D: int):
    b_per_w = B // NUM_WORKERS
    n_chunks = b_per_w // CHUNK
    mesh = plsc.VectorSubcoreMesh(core_axis_name="c", subcore_axis_name="s")

    @functools.partial(
        pl.kernel,
        mesh=mesh,
        out_type=jax.ShapeDtypeStruct((B, D), jnp.float32),
        scratch_types=(
            [pltpu.VMEM((b_per_w,), jnp.int32)]
            + [pltpu.VMEM((CHUNK, D), jnp.float32)] * NBUF
            + [pltpu.SemaphoreType.DMA] * (2 * NBUF)
        ),
    )
    def gather_kernel(pos_hbm, table_hbm, out_hbm, idx_v, *rest):
        bufs = rest[:NBUF]
        gsems = rest[NBUF:2 * NBUF]
        wsems = rest[2 * NBUF:]
        wid = lax.axis_index("s") * NUM_CORES + lax.axis_index("c")
        base = wid * b_per_w

        pltpu.sync_copy(pos_hbm.at[pl.ds(base, b_per_w)], idx_v)

        def gather_desc(j, b):
            return pltpu.make_async_copy(
                table_hbm.at[idx_v.at[pl.ds(j * CHUNK, CHUNK)]],
                bufs[b], gsems[b])

        def wb_desc(j, b):
            return pltpu.make_async_copy(
                bufs[b], out_hbm.at[pl.ds(base + j * CHUNK, CHUNK)],
                wsems[b])

        for b in range(NBUF):
            gather_desc(b, b).start()

        def step(j, b):
            gather_desc(j, b).wait()
            wb_desc(j, b).start()
            wb_desc(j, b).wait()
            gather_desc(j + NBUF, b).start()

        main = n_chunks - NBUF
        unrolled = (main // NBUF) * NBUF

        def body(k, carry):
            for b in range(NBUF):
                step(k * NBUF + b, b)
            return carry

        lax.fori_loop(0, main // NBUF, body, 0)

        for j in range(unrolled, main):
            step(j, j % NBUF)

        for j in range(main, n_chunks):
            gather_desc(j, j % NBUF).wait()
            wb_desc(j, j % NBUF).start()
        for j in range(main, n_chunks):
            wb_desc(j, j % NBUF).wait()

    return gather_kernel


def kernel(pos, table):
    V, D = table.shape
    flat_pos = pos.reshape(-1).astype(jnp.int32)
    B = flat_pos.shape[0]
    out = _make_gather(B, V, D)(flat_pos, table)
    return out.reshape(pos.shape + (D,))
